# baseline (speedup 1.0000x reference)
import jax
import jax.numpy as jnp
from jax import lax
from jax.experimental import pallas as pl
from jax.experimental.pallas import tpu as pltpu
from jax.experimental.pallas import tpu_sc as plsc

_N = 262144
_R = 128
_R3 = _R * _R * _R
_NCH = 27
_PITCH = 32
_VPITCH = 33
_NC, _NS, _L = 2, 16, 16
_NW = _NC * _NS
_P = _N // _NW
_C = 128
_NCHUNK = _P // _C
_NG = _C // _L
_NIDX = 8 * _C
_GSZ = 128
_NDMA = _NIDX // _GSZ

_SH_C = (0.282095, 0.488603, 1.092548, 0.315392, 0.546274)


def _rsqrt(x):
    i = lax.bitcast_convert_type(x, jnp.int32)
    i = jnp.int32(0x5F3759DF) - (i >> 1)
    y = lax.bitcast_convert_type(i, jnp.float32)
    for _ in range(3):
        y = y * (1.5 - 0.5 * x * y * y)
    return y


def _to_grid(v):
    n = jnp.clip((v + 1.5) * (2.0 / 3.0) - 1.0, -1.0, 1.0)
    return (n + 1.0) * (0.5 * (_R - 1))


def _floor_i(p):
    i = p.astype(jnp.int32)
    return i - jnp.where(i.astype(jnp.float32) > p, 1, 0)


_BT = 2048
_SPITCH = _BT + 1
_XPW = _R // _NW


def _build_body(shp, dens, stab_out, stage, outb, sem):
    c = lax.axis_index("c")
    s = lax.axis_index("s")
    wid = s * _NC + c
    lanes = lax.iota(jnp.int32, _L)
    hirow = _L + jnp.minimum(lanes, _NCH - _L)
    himask = lanes < (_NCH + 1 - _L)

    @pl.loop(0, _XPW * (_R * _R // _BT))
    def _tile(i):
        xi = wid * _XPW + i // (_R * _R // _BT)
        off = (i % (_R * _R // _BT)) * _BT
        for ch in range(_NCH):
            pltpu.async_copy(shp.at[xi, ch, pl.ds(off, _BT)],
                             stage.at[ch, pl.ds(0, _BT)], sem)
        pltpu.async_copy(dens.at[xi, pl.ds(off, _BT)],
                         stage.at[_NCH, pl.ds(0, _BT)], sem)
        for ch in range(_NCH):
            pltpu.make_async_copy(shp.at[xi, ch, pl.ds(off, _BT)],
                                  stage.at[ch, pl.ds(0, _BT)], sem).wait()
        pltpu.make_async_copy(dens.at[xi, pl.ds(off, _BT)],
                              stage.at[_NCH, pl.ds(0, _BT)], sem).wait()

        @pl.loop(0, _BT, unroll=8)
        def _pt(pt):
            ptv = lanes * 0 + pt
            lo = plsc.load_gather(stage, [lanes, ptv])
            hi = plsc.load_gather(stage, [hirow, ptv])
            hi = jnp.where(himask, hi, 0.0)
            outb[pt, pl.ds(0, _L)] = lo
            outb[pt, pl.ds(_L, _L)] = hi

        pltpu.sync_copy(outb, stab_out.at[pl.ds(xi * (_R * _R) + off, _BT), :])


def _tec_body(ptst, dirst, stab, dout, cout,
              xb0, yb0, zb0, vxb0, vyb0, vzb0, idxb0, wb0, shr0,
              xb1, yb1, zb1, vxb1, vyb1, vzb1, idxb1, wb1, shr1,
              colb, dob, trb, sem0, sem1):
    c = lax.axis_index("c")
    s = lax.axis_index("s")
    wid = s * _NC + c
    base_w = wid * _P
    lanes = lax.iota(jnp.int32, _L)
    xb = (xb0, xb1)
    yb = (yb0, yb1)
    zb = (zb0, zb1)
    vxb = (vxb0, vxb1)
    vyb = (vyb0, vyb1)
    vzb = (vzb0, vzb1)
    idxb = (idxb0, idxb1)
    wb = (wb0, wb1)
    shr = (shr0, shr1)
    sem = (sem0, sem1)

    def prep(t, p):
        base = base_w + t * _C
        pltpu.sync_copy(ptst.at[0, pl.ds(base, _C)], xb[p])
        pltpu.sync_copy(ptst.at[1, pl.ds(base, _C)], yb[p])
        pltpu.sync_copy(ptst.at[2, pl.ds(base, _C)], zb[p])
        pltpu.sync_copy(dirst.at[0, pl.ds(base, _C)], vxb[p])
        pltpu.sync_copy(dirst.at[1, pl.ds(base, _C)], vyb[p])
        pltpu.sync_copy(dirst.at[2, pl.ds(base, _C)], vzb[p])

        @pl.loop(0, _NG)
        def _grp(g):
            o = g * _L
            px = _to_grid(xb[p][pl.ds(o, _L)])
            py = _to_grid(yb[p][pl.ds(o, _L)])
            pz = _to_grid(zb[p][pl.ds(o, _L)])
            xi = _floor_i(px)
            yi = _floor_i(py)
            zi = _floor_i(pz)
            fx = px - xi.astype(jnp.float32)
            fy = py - yi.astype(jnp.float32)
            fz = pz - zi.astype(jnp.float32)
            dxs = (jnp.minimum(xi + 1, _R - 1) - xi) * (_R * _R)
            dys = (jnp.minimum(yi + 1, _R - 1) - yi) * _R
            dzs = jnp.minimum(zi + 1, _R - 1) - zi
            b0 = xi * (_R * _R) + yi * _R + zi
            gx = (1.0 - fx, fx)
            gy = (1.0 - fy, fy)
            gz = (1.0 - fz, fz)
            k = 0
            for a in (0, 1):
                ia = b0 + dxs if a else b0
                for b in (0, 1):
                    iab = ia + dys if b else ia
                    wab = gx[a] * gy[b]
                    for cc in (0, 1):
                        idx = iab + dzs if cc else iab
                        idxb[p][pl.ds(k * _C + o, _L)] = idx
                        wb[p][pl.ds(k * _C + o, _L)] = wab * gz[cc]
                        k += 1

        for j in range(_NDMA):
            pltpu.async_copy(
                stab.at[idxb[p].at[pl.ds(j * _GSZ, _GSZ)]],
                shr[p].at[pl.ds(j * _GSZ, _GSZ), :], sem[p])

    def finish(t, p):
        base = base_w + t * _C
        for j in range(_NDMA):
            pltpu.make_async_copy(
                stab.at[pl.ds(0, _GSZ), :],
                shr[p].at[pl.ds(j * _GSZ, _GSZ), :], sem[p]).wait()

        @pl.loop(0, _NG)
        def _comb(g):
            o = g * _L
            ws = [wb[p][pl.ds(k * _C + o, _L)] for k in range(8)]
            for pt in range(_L):
                row_acc = [None, None]
                for k in range(8):
                    rr = k * _C + o + pt
                    wk = ws[k][pt]
                    for h in range(2):
                        v = wk * shr[p][rr, pl.ds(h * _L, _L)]
                        row_acc[h] = v if row_acc[h] is None else row_acc[h] + v
                plsc.store_scatter(trb, [lanes * 17 + pt], row_acc[0])
                plsc.store_scatter(trb, [(lanes + _L) * 17 + pt], row_acc[1])

            def interp(ch):
                return trb[pl.ds(ch * 17, _L)]

            dob[pl.ds(o, _L)] = jnp.maximum(interp(_NCH), 0.0)

            vx = vxb[p][pl.ds(o, _L)]
            vy = vyb[p][pl.ds(o, _L)]
            vz = vzb[p][pl.ds(o, _L)]
            sq = vx * vx + vy * vy + vz * vz
            m = sq < 1e-8
            vx = jnp.where(m, 0.0, vx)
            vy = jnp.where(m, 0.0, vy)
            vz = jnp.where(m, 1.0, vz)
            sqn = jnp.where(m, 1.0, sq)
            r = _rsqrt(sqn)
            bx = vx * r
            by = vy * r
            bz = vz * r
            c0, c1, c2, c3, c4 = _SH_C
            basis = (
                jnp.full((_L,), c0, jnp.float32),
                c1 * by, c1 * bz, c1 * bx,
                c2 * bx * by, c2 * by * bz,
                c3 * (3.0 * bz * bz - 1.0),
                c2 * bx * bz,
                c4 * (bx * bx - by * by),
            )
            for j3 in range(3):
                acc = None
                for s9 in range(9):
                    tb = basis[s9] * interp(j3 * 9 + s9)
                    acc = tb if acc is None else acc + tb
                colv = 1.0 / (1.0 + jnp.exp(-acc))
                plsc.store_scatter(colb, [(o + lanes) * 3 + j3], colv)

        pltpu.sync_copy(dob, dout.at[pl.ds(base, _C)])
        pltpu.sync_copy(colb, cout.at[pl.ds(base * 3, _C * 3)])

    prep(0, 0)

    @pl.loop(0, _NCHUNK // 2 - 1)
    def _steady(i):
        t0 = 2 * i
        prep(t0 + 1, 1)
        finish(t0, 0)
        prep(t0 + 2, 0)
        finish(t0 + 1, 1)

    prep(_NCHUNK - 1, 1)
    finish(_NCHUNK - 2, 0)
    finish(_NCHUNK - 1, 1)


def kernel(points, view_dirs, density_grid, sh_grid):
    ptst = points.T
    dirst = view_dirs.T
    shp = jnp.transpose(sh_grid, (0, 3, 4, 1, 2)).reshape(_R, _NCH, _R * _R)
    dens2 = density_grid.reshape(_R, _R * _R)

    mesh = plsc.VectorSubcoreMesh(
        core_axis_name="c", subcore_axis_name="s",
        num_cores=_NC, num_subcores=_NS)
    build = pl.kernel(
        _build_body,
        out_type=jax.ShapeDtypeStruct((_R3, _PITCH), jnp.float32),
        mesh=mesh,
        compiler_params=pltpu.CompilerParams(
            needs_layout_passes=False, use_tc_tiling_on_sc=False),
        scratch_types=[
            pltpu.VMEM((_NCH + 1, _SPITCH), jnp.float32),
            pltpu.VMEM((_BT, _PITCH), jnp.float32),
            pltpu.SemaphoreType.DMA,
        ],
    )
    stab = build(shp, dens2)
    dbuf = [
        pltpu.VMEM((_C,), jnp.float32),
        pltpu.VMEM((_C,), jnp.float32),
        pltpu.VMEM((_C,), jnp.float32),
        pltpu.VMEM((_C,), jnp.float32),
        pltpu.VMEM((_C,), jnp.float32),
        pltpu.VMEM((_C,), jnp.float32),
        pltpu.VMEM((_NIDX,), jnp.int32),
        pltpu.VMEM((_NIDX,), jnp.float32),
        pltpu.VMEM((_NIDX, _PITCH), jnp.float32),
    ]
    call = pl.kernel(
        _tec_body,
        out_type=(
            jax.ShapeDtypeStruct((_N,), jnp.float32),
            jax.ShapeDtypeStruct((_N * 3,), jnp.float32),
        ),
        mesh=mesh,
        compiler_params=pltpu.CompilerParams(
            needs_layout_passes=False, use_tc_tiling_on_sc=False),
        scratch_types=dbuf + dbuf + [
            pltpu.VMEM((_C * 3,), jnp.float32),
            pltpu.VMEM((_C,), jnp.float32),
            pltpu.VMEM((2 * _L * 17,), jnp.float32),
            pltpu.SemaphoreType.DMA,
            pltpu.SemaphoreType.DMA,
        ],
    )
    density, colors = call(ptst, dirst, stab)
    return density, colors.reshape(_N, 3)


# speedup vs baseline: 1.0742x; 1.0742x over previous
import jax
import jax.numpy as jnp
from jax import lax
from jax.experimental import pallas as pl
from jax.experimental.pallas import tpu as pltpu
from jax.experimental.pallas import tpu_sc as plsc

_N = 262144
_R = 128
_R3 = _R * _R * _R
_NCH = 27
_PITCH = 32
_VPITCH = 33
_NC, _NS, _L = 2, 16, 16
_NW = _NC * _NS
_P = _N // _NW
_C = 128
_NCHUNK = _P // _C
_NG = _C // _L
_NIDX = 8 * _C
_GSZ = 128
_NDMA = _NIDX // _GSZ

_SH_C = (0.282095, 0.488603, 1.092548, 0.315392, 0.546274)


def _rsqrt(x):
    i = lax.bitcast_convert_type(x, jnp.int32)
    i = jnp.int32(0x5F3759DF) - (i >> 1)
    y = lax.bitcast_convert_type(i, jnp.float32)
    for _ in range(3):
        y = y * (1.5 - 0.5 * x * y * y)
    return y


def _to_grid(v):
    n = jnp.clip((v + 1.5) * (2.0 / 3.0) - 1.0, -1.0, 1.0)
    return (n + 1.0) * (0.5 * (_R - 1))


def _floor_i(p):
    i = p.astype(jnp.int32)
    return i - jnp.where(i.astype(jnp.float32) > p, 1, 0)


_BT = 1024
_SPITCH = _BT + 1
_XPW = _R // _NW
_TPX = _R * _R // _BT
_NT = _XPW * _TPX


def _build_body(shp, dens, stab_out, stage0, stage1, outb, bsem0, bsem1):
    c = lax.axis_index("c")
    s = lax.axis_index("s")
    wid = s * _NC + c
    lanes = lax.iota(jnp.int32, _L)
    hirow = _L + jnp.minimum(lanes, _NCH - _L)
    himask = lanes < (_NCH + 1 - _L)
    stage = (stage0, stage1)
    sem = (bsem0, bsem1)

    def fire(t, p):
        xi = wid * _XPW + t // _TPX
        off = (t % _TPX) * _BT
        for ch in range(_NCH):
            pltpu.async_copy(shp.at[xi, ch, pl.ds(off, _BT)],
                             stage[p].at[ch, pl.ds(0, _BT)], sem[p])
        pltpu.async_copy(dens.at[xi, pl.ds(off, _BT)],
                         stage[p].at[_NCH, pl.ds(0, _BT)], sem[p])

    def compute(t, p):
        xi = wid * _XPW + t // _TPX
        off = (t % _TPX) * _BT
        for ch in range(_NCH + 1):
            pltpu.make_async_copy(dens.at[0, pl.ds(0, _BT)],
                                  stage[p].at[ch, pl.ds(0, _BT)], sem[p]).wait()

        @pl.loop(0, _BT)
        def _pt(pt):
            ptv = lanes * 0 + pt
            lo = plsc.load_gather(stage[p], [lanes, ptv])
            hi = plsc.load_gather(stage[p], [hirow, ptv])
            hi = jnp.where(himask, hi, 0.0)
            outb[pt, pl.ds(0, _L)] = lo
            outb[pt, pl.ds(_L, _L)] = hi

        pltpu.sync_copy(outb, stab_out.at[pl.ds(xi * (_R * _R) + off, _BT), :])

    fire(0, 0)

    @pl.loop(0, _NT // 2 - 1)
    def _steady(j):
        t0 = 2 * j
        fire(t0 + 1, 1)
        compute(t0, 0)
        fire(t0 + 2, 0)
        compute(t0 + 1, 1)

    fire(_NT - 1, 1)
    compute(_NT - 2, 0)
    compute(_NT - 1, 1)


def _tec_body(ptst, dirst, stab, dout, cout,
              xb0, yb0, zb0, vxb0, vyb0, vzb0, idxb0, wb0, shr0,
              xb1, yb1, zb1, vxb1, vyb1, vzb1, idxb1, wb1, shr1,
              colb, dob, trb, sem0, sem1):
    c = lax.axis_index("c")
    s = lax.axis_index("s")
    wid = s * _NC + c
    base_w = wid * _P
    lanes = lax.iota(jnp.int32, _L)
    xb = (xb0, xb1)
    yb = (yb0, yb1)
    zb = (zb0, zb1)
    vxb = (vxb0, vxb1)
    vyb = (vyb0, vyb1)
    vzb = (vzb0, vzb1)
    idxb = (idxb0, idxb1)
    wb = (wb0, wb1)
    shr = (shr0, shr1)
    sem = (sem0, sem1)

    def prep(t, p):
        base = base_w + t * _C
        pltpu.sync_copy(ptst.at[0, pl.ds(base, _C)], xb[p])
        pltpu.sync_copy(ptst.at[1, pl.ds(base, _C)], yb[p])
        pltpu.sync_copy(ptst.at[2, pl.ds(base, _C)], zb[p])
        pltpu.sync_copy(dirst.at[0, pl.ds(base, _C)], vxb[p])
        pltpu.sync_copy(dirst.at[1, pl.ds(base, _C)], vyb[p])
        pltpu.sync_copy(dirst.at[2, pl.ds(base, _C)], vzb[p])

        @pl.loop(0, _NG)
        def _grp(g):
            o = g * _L
            px = _to_grid(xb[p][pl.ds(o, _L)])
            py = _to_grid(yb[p][pl.ds(o, _L)])
            pz = _to_grid(zb[p][pl.ds(o, _L)])
            xi = _floor_i(px)
            yi = _floor_i(py)
            zi = _floor_i(pz)
            fx = px - xi.astype(jnp.float32)
            fy = py - yi.astype(jnp.float32)
            fz = pz - zi.astype(jnp.float32)
            dxs = (jnp.minimum(xi + 1, _R - 1) - xi) * (_R * _R)
            dys = (jnp.minimum(yi + 1, _R - 1) - yi) * _R
            dzs = jnp.minimum(zi + 1, _R - 1) - zi
            b0 = xi * (_R * _R) + yi * _R + zi
            gx = (1.0 - fx, fx)
            gy = (1.0 - fy, fy)
            gz = (1.0 - fz, fz)
            k = 0
            for a in (0, 1):
                ia = b0 + dxs if a else b0
                for b in (0, 1):
                    iab = ia + dys if b else ia
                    wab = gx[a] * gy[b]
                    for cc in (0, 1):
                        idx = iab + dzs if cc else iab
                        idxb[p][pl.ds(k * _C + o, _L)] = idx
                        wb[p][pl.ds(k * _C + o, _L)] = wab * gz[cc]
                        k += 1

        for j in range(_NDMA):
            pltpu.async_copy(
                stab.at[idxb[p].at[pl.ds(j * _GSZ, _GSZ)]],
                shr[p].at[pl.ds(j * _GSZ, _GSZ), :], sem[p])

    def finish(t, p):
        base = base_w + t * _C
        for j in range(_NDMA):
            pltpu.make_async_copy(
                stab.at[pl.ds(0, _GSZ), :],
                shr[p].at[pl.ds(j * _GSZ, _GSZ), :], sem[p]).wait()

        @pl.loop(0, _NG)
        def _comb(g):
            o = g * _L
            ws = [wb[p][pl.ds(k * _C + o, _L)] for k in range(8)]
            for pt in range(_L):
                row_acc = [None, None]
                for k in range(8):
                    rr = k * _C + o + pt
                    wk = ws[k][pt]
                    for h in range(2):
                        v = wk * shr[p][rr, pl.ds(h * _L, _L)]
                        row_acc[h] = v if row_acc[h] is None else row_acc[h] + v
                plsc.store_scatter(trb, [lanes * 17 + pt], row_acc[0])
                plsc.store_scatter(trb, [(lanes + _L) * 17 + pt], row_acc[1])

            def interp(ch):
                return trb[pl.ds(ch * 17, _L)]

            dob[pl.ds(o, _L)] = jnp.maximum(interp(_NCH), 0.0)

            vx = vxb[p][pl.ds(o, _L)]
            vy = vyb[p][pl.ds(o, _L)]
            vz = vzb[p][pl.ds(o, _L)]
            sq = vx * vx + vy * vy + vz * vz
            m = sq < 1e-8
            vx = jnp.where(m, 0.0, vx)
            vy = jnp.where(m, 0.0, vy)
            vz = jnp.where(m, 1.0, vz)
            sqn = jnp.where(m, 1.0, sq)
            r = _rsqrt(sqn)
            bx = vx * r
            by = vy * r
            bz = vz * r
            c0, c1, c2, c3, c4 = _SH_C
            basis = (
                jnp.full((_L,), c0, jnp.float32),
                c1 * by, c1 * bz, c1 * bx,
                c2 * bx * by, c2 * by * bz,
                c3 * (3.0 * bz * bz - 1.0),
                c2 * bx * bz,
                c4 * (bx * bx - by * by),
            )
            for j3 in range(3):
                acc = None
                for s9 in range(9):
                    tb = basis[s9] * interp(j3 * 9 + s9)
                    acc = tb if acc is None else acc + tb
                colv = 1.0 / (1.0 + jnp.exp(-acc))
                plsc.store_scatter(colb, [(o + lanes) * 3 + j3], colv)

        pltpu.sync_copy(dob, dout.at[pl.ds(base, _C)])
        pltpu.sync_copy(colb, cout.at[pl.ds(base * 3, _C * 3)])

    prep(0, 0)

    @pl.loop(0, _NCHUNK // 2 - 1)
    def _steady(i):
        t0 = 2 * i
        prep(t0 + 1, 1)
        finish(t0, 0)
        prep(t0 + 2, 0)
        finish(t0 + 1, 1)

    prep(_NCHUNK - 1, 1)
    finish(_NCHUNK - 2, 0)
    finish(_NCHUNK - 1, 1)


def kernel(points, view_dirs, density_grid, sh_grid):
    ptst = points.T
    dirst = view_dirs.T
    shp = jnp.transpose(sh_grid, (0, 3, 4, 1, 2)).reshape(_R, _NCH, _R * _R)
    dens2 = density_grid.reshape(_R, _R * _R)

    mesh = plsc.VectorSubcoreMesh(
        core_axis_name="c", subcore_axis_name="s",
        num_cores=_NC, num_subcores=_NS)
    build = pl.kernel(
        _build_body,
        out_type=jax.ShapeDtypeStruct((_R3, _PITCH), jnp.float32),
        mesh=mesh,
        compiler_params=pltpu.CompilerParams(
            needs_layout_passes=False, use_tc_tiling_on_sc=False),
        scratch_types=[
            pltpu.VMEM((_NCH + 1, _SPITCH), jnp.float32),
            pltpu.VMEM((_NCH + 1, _SPITCH), jnp.float32),
            pltpu.VMEM((_BT, _PITCH), jnp.float32),
            pltpu.SemaphoreType.DMA,
            pltpu.SemaphoreType.DMA,
        ],
    )
    stab = build(shp, dens2)
    dbuf = [
        pltpu.VMEM((_C,), jnp.float32),
        pltpu.VMEM((_C,), jnp.float32),
        pltpu.VMEM((_C,), jnp.float32),
        pltpu.VMEM((_C,), jnp.float32),
        pltpu.VMEM((_C,), jnp.float32),
        pltpu.VMEM((_C,), jnp.float32),
        pltpu.VMEM((_NIDX,), jnp.int32),
        pltpu.VMEM((_NIDX,), jnp.float32),
        pltpu.VMEM((_NIDX, _PITCH), jnp.float32),
    ]
    call = pl.kernel(
        _tec_body,
        out_type=(
            jax.ShapeDtypeStruct((_N,), jnp.float32),
            jax.ShapeDtypeStruct((_N * 3,), jnp.float32),
        ),
        mesh=mesh,
        compiler_params=pltpu.CompilerParams(
            needs_layout_passes=False, use_tc_tiling_on_sc=False),
        scratch_types=dbuf + dbuf + [
            pltpu.VMEM((_C * 3,), jnp.float32),
            pltpu.VMEM((_C,), jnp.float32),
            pltpu.VMEM((2 * _L * 17,), jnp.float32),
            pltpu.SemaphoreType.DMA,
            pltpu.SemaphoreType.DMA,
        ],
    )
    density, colors = call(ptst, dirst, stab)
    return density, colors.reshape(_N, 3)


# speedup vs baseline: 1.0743x; 1.0001x over previous
import jax
import jax.numpy as jnp
from jax import lax
from jax.experimental import pallas as pl
from jax.experimental.pallas import tpu as pltpu
from jax.experimental.pallas import tpu_sc as plsc

_N = 262144
_R = 128
_R3 = _R * _R * _R
_NCH = 27
_PITCH = 32
_NC, _NS, _L = 2, 16, 16
_NW = _NC * _NS
_P = _N // _NW
_C = 128
_NCHUNK = _P // _C
_NG = _C // _L
_NIDX = 8 * _C
_GSZ = 128
_NDMA = _NIDX // _GSZ

_SH_C = (0.282095, 0.488603, 1.092548, 0.315392, 0.546274)


def _rsqrt(x):
    i = lax.bitcast_convert_type(x, jnp.int32)
    i = jnp.int32(0x5F3759DF) - (i >> 1)
    y = lax.bitcast_convert_type(i, jnp.float32)
    for _ in range(3):
        y = y * (1.5 - 0.5 * x * y * y)
    return y


def _to_grid(v):
    n = jnp.clip((v + 1.5) * (2.0 / 3.0) - 1.0, -1.0, 1.0)
    return (n + 1.0) * (0.5 * (_R - 1))


def _floor_i(p):
    i = p.astype(jnp.int32)
    return i - jnp.where(i.astype(jnp.float32) > p, 1, 0)


_BT = 1024
_SPITCH = _BT + 1
_XPW = _R // _NW
_TPX = _R * _R // _BT
_NT = _XPW * _TPX


def _build_body(shp, dens, stab_out, stage0, stage1, outb, bsem0, bsem1):
    c = lax.axis_index("c")
    s = lax.axis_index("s")
    wid = s * _NC + c
    lanes = lax.iota(jnp.int32, _L)
    hirow = _L + jnp.minimum(lanes, _NCH - _L)
    himask = lanes < (_NCH + 1 - _L)
    stage = (stage0, stage1)
    sem = (bsem0, bsem1)

    def fire(t, p):
        xi = wid * _XPW + t // _TPX
        off = (t % _TPX) * _BT
        for ch in range(_NCH):
            pltpu.async_copy(shp.at[xi, ch, pl.ds(off, _BT)],
                             stage[p].at[ch, pl.ds(0, _BT)], sem[p])
        pltpu.async_copy(dens.at[xi, pl.ds(off, _BT)],
                         stage[p].at[_NCH, pl.ds(0, _BT)], sem[p])

    def compute(t, p):
        xi = wid * _XPW + t // _TPX
        off = (t % _TPX) * _BT
        for ch in range(_NCH + 1):
            pltpu.make_async_copy(dens.at[0, pl.ds(0, _BT)],
                                  stage[p].at[ch, pl.ds(0, _BT)], sem[p]).wait()

        @pl.loop(0, _BT)
        def _pt(pt):
            ptv = lanes * 0 + pt
            lo = plsc.load_gather(stage[p], [lanes, ptv])
            hi = plsc.load_gather(stage[p], [hirow, ptv])
            hi = jnp.where(himask, hi, 0.0)
            outb[pt, pl.ds(0, _L)] = lo
            outb[pt, pl.ds(_L, _L)] = hi

        pltpu.sync_copy(outb, stab_out.at[pl.ds(xi * (_R * _R) + off, _BT), :])

    fire(0, 0)

    @pl.loop(0, _NT // 2 - 1)
    def _steady(j):
        t0 = 2 * j
        fire(t0 + 1, 1)
        compute(t0, 0)
        fire(t0 + 2, 0)
        compute(t0 + 1, 1)

    fire(_NT - 1, 1)
    compute(_NT - 2, 0)
    compute(_NT - 1, 1)


def _tec_body(ptst, dirst, stab, dout, cout,
              xb0, yb0, zb0, vxb0, vyb0, vzb0, idxb0, wb0, shr0,
              xb1, yb1, zb1, vxb1, vyb1, vzb1, idxb1, wb1, shr1,
              colb, dob, trb, sem0, sem1):
    c = lax.axis_index("c")
    s = lax.axis_index("s")
    wid = s * _NC + c
    base_w = wid * _P
    lanes = lax.iota(jnp.int32, _L)
    xb = (xb0, xb1)
    yb = (yb0, yb1)
    zb = (zb0, zb1)
    vxb = (vxb0, vxb1)
    vyb = (vyb0, vyb1)
    vzb = (vzb0, vzb1)
    idxb = (idxb0, idxb1)
    wb = (wb0, wb1)
    shr = (shr0, shr1)
    sem = (sem0, sem1)

    def prep(t, p):
        base = base_w + t * _C
        pltpu.sync_copy(ptst.at[0, pl.ds(base, _C)], xb[p])
        pltpu.sync_copy(ptst.at[1, pl.ds(base, _C)], yb[p])
        pltpu.sync_copy(ptst.at[2, pl.ds(base, _C)], zb[p])
        pltpu.sync_copy(dirst.at[0, pl.ds(base, _C)], vxb[p])
        pltpu.sync_copy(dirst.at[1, pl.ds(base, _C)], vyb[p])
        pltpu.sync_copy(dirst.at[2, pl.ds(base, _C)], vzb[p])

        @pl.loop(0, _NG)
        def _grp(g):
            o = g * _L
            px = _to_grid(xb[p][pl.ds(o, _L)])
            py = _to_grid(yb[p][pl.ds(o, _L)])
            pz = _to_grid(zb[p][pl.ds(o, _L)])
            xi = _floor_i(px)
            yi = _floor_i(py)
            zi = _floor_i(pz)
            fx = px - xi.astype(jnp.float32)
            fy = py - yi.astype(jnp.float32)
            fz = pz - zi.astype(jnp.float32)
            dxs = (jnp.minimum(xi + 1, _R - 1) - xi) * (_R * _R)
            dys = (jnp.minimum(yi + 1, _R - 1) - yi) * _R
            dzs = jnp.minimum(zi + 1, _R - 1) - zi
            b0 = xi * (_R * _R) + yi * _R + zi
            gx = (1.0 - fx, fx)
            gy = (1.0 - fy, fy)
            gz = (1.0 - fz, fz)
            k = 0
            for a in (0, 1):
                ia = b0 + dxs if a else b0
                for b in (0, 1):
                    iab = ia + dys if b else ia
                    wab = gx[a] * gy[b]
                    for cc in (0, 1):
                        idx = iab + dzs if cc else iab
                        idxb[p][pl.ds(k * _C + o, _L)] = idx
                        wb[p][pl.ds(k * _C + o, _L)] = wab * gz[cc]
                        k += 1

        for j in range(_NDMA):
            pltpu.async_copy(
                stab.at[idxb[p].at[pl.ds(j * _GSZ, _GSZ)]],
                shr[p].at[pl.ds(j * _GSZ, _GSZ), :], sem[p])

    def finish(t, p):
        base = base_w + t * _C
        for j in range(_NDMA):
            pltpu.make_async_copy(
                stab.at[pl.ds(0, _GSZ), :],
                shr[p].at[pl.ds(j * _GSZ, _GSZ), :], sem[p]).wait()

        @pl.loop(0, _NG)
        def _comb(g):
            o = g * _L
            ws = [wb[p][pl.ds(k * _C + o, _L)] for k in range(8)]
            for pt in range(_L):
                row_acc = [None, None]
                for k in range(8):
                    rr = k * _C + o + pt
                    wk = ws[k][pt]
                    for h in range(2):
                        v = wk * shr[p][rr, pl.ds(h * _L, _L)]
                        row_acc[h] = v if row_acc[h] is None else row_acc[h] + v
                plsc.store_scatter(trb, [lanes * 17 + pt], row_acc[0])
                plsc.store_scatter(trb, [(lanes + _L) * 17 + pt], row_acc[1])

            def interp(ch):
                return trb[pl.ds(ch * 17, _L)]

            dob[pl.ds(o, _L)] = jnp.maximum(interp(_NCH), 0.0)

            vx = vxb[p][pl.ds(o, _L)]
            vy = vyb[p][pl.ds(o, _L)]
            vz = vzb[p][pl.ds(o, _L)]
            sq = vx * vx + vy * vy + vz * vz
            m = sq < 1e-8
            vx = jnp.where(m, 0.0, vx)
            vy = jnp.where(m, 0.0, vy)
            vz = jnp.where(m, 1.0, vz)
            sqn = jnp.where(m, 1.0, sq)
            r = _rsqrt(sqn)
            bx = vx * r
            by = vy * r
            bz = vz * r
            c0, c1, c2, c3, c4 = _SH_C
            basis = (
                jnp.full((_L,), c0, jnp.float32),
                c1 * by, c1 * bz, c1 * bx,
                c2 * bx * by, c2 * by * bz,
                c3 * (3.0 * bz * bz - 1.0),
                c2 * bx * bz,
                c4 * (bx * bx - by * by),
            )
            for j3 in range(3):
                acc = None
                for s9 in range(9):
                    tb = basis[s9] * interp(j3 * 9 + s9)
                    acc = tb if acc is None else acc + tb
                colv = 1.0 / (1.0 + jnp.exp(-acc))
                plsc.store_scatter(colb, [(o + lanes) * 3 + j3], colv)

        pltpu.sync_copy(dob, dout.at[pl.ds(base, _C)])
        pltpu.sync_copy(colb, cout.at[pl.ds(base * 3, _C * 3)])

    prep(0, 0)

    @pl.loop(0, _NCHUNK // 2 - 1)
    def _steady(i):
        t0 = 2 * i
        prep(t0 + 1, 1)
        finish(t0, 0)
        prep(t0 + 2, 0)
        finish(t0 + 1, 1)

    prep(_NCHUNK - 1, 1)
    finish(_NCHUNK - 2, 0)
    finish(_NCHUNK - 1, 1)


def kernel(points, view_dirs, density_grid, sh_grid):
    ptst = points.T
    dirst = view_dirs.T
    shp = jnp.transpose(sh_grid, (0, 3, 4, 1, 2)).reshape(_R, _NCH, _R * _R)
    dens2 = density_grid.reshape(_R, _R * _R)

    mesh = plsc.VectorSubcoreMesh(
        core_axis_name="c", subcore_axis_name="s",
        num_cores=_NC, num_subcores=_NS)
    build = pl.kernel(
        _build_body,
        out_type=jax.ShapeDtypeStruct((_R3, _PITCH), jnp.float32),
        mesh=mesh,
        compiler_params=pltpu.CompilerParams(
            needs_layout_passes=False, use_tc_tiling_on_sc=False),
        scratch_types=[
            pltpu.VMEM((_NCH + 1, _SPITCH), jnp.float32),
            pltpu.VMEM((_NCH + 1, _SPITCH), jnp.float32),
            pltpu.VMEM((_BT, _PITCH), jnp.float32),
            pltpu.SemaphoreType.DMA,
            pltpu.SemaphoreType.DMA,
        ],
    )
    stab = build(shp, dens2)
    dbuf = [
        pltpu.VMEM((_C,), jnp.float32),
        pltpu.VMEM((_C,), jnp.float32),
        pltpu.VMEM((_C,), jnp.float32),
        pltpu.VMEM((_C,), jnp.float32),
        pltpu.VMEM((_C,), jnp.float32),
        pltpu.VMEM((_C,), jnp.float32),
        pltpu.VMEM((_NIDX,), jnp.int32),
        pltpu.VMEM((_NIDX,), jnp.float32),
        pltpu.VMEM((_NIDX, _PITCH), jnp.float32),
    ]
    call = pl.kernel(
        _tec_body,
        out_type=(
            jax.ShapeDtypeStruct((_N,), jnp.float32),
            jax.ShapeDtypeStruct((_N * 3,), jnp.float32),
        ),
        mesh=mesh,
        compiler_params=pltpu.CompilerParams(
            needs_layout_passes=False, use_tc_tiling_on_sc=False),
        scratch_types=dbuf + dbuf + [
            pltpu.VMEM((_C * 3,), jnp.float32),
            pltpu.VMEM((_C,), jnp.float32),
            pltpu.VMEM((2 * _L * 17,), jnp.float32),
            pltpu.SemaphoreType.DMA,
            pltpu.SemaphoreType.DMA,
        ],
    )
    density, colors = call(ptst, dirst, stab)
    return density, colors.reshape(_N, 3)
